# baseline (speedup 1.0000x reference)
import functools
import math

import jax
import jax.numpy as jnp
from jax import lax
from jax.experimental import pallas as pl
from jax.experimental.pallas import tpu as pltpu


def _gem_body(p_ref, x_ref, o_ref, *, eps, log_hw, hw, g):
    p = p_ref[0]
    x = x_ref[...]
    xc = jnp.maximum(x, eps)
    xp = jnp.exp(p * jnp.log(xc))
    w = lax.broadcasted_iota(jnp.int32, (g * hw, g), 0)
    j = lax.broadcasted_iota(jnp.int32, (g * hw, g), 1)
    seg = (w // hw == j).astype(jnp.float32)
    s = jax.lax.dot_general(xp, seg, (((1,), (0,)), ((), ())),
                            preferred_element_type=jnp.float32)
    o_ref[...] = jnp.exp((jnp.log(s) - log_hw) / p)


def kernel(x, p):
    N, C, H, W = x.shape
    NC, HW = N * C, H * W
    G = max(1, 128 // HW)
    while NC % G:
        G //= 2
    R = NC // G
    x2d = x.reshape(R, G * HW)

    RB = 2048
    while R % RB:
        RB //= 2
    nb = R // RB

    p_arr = jnp.asarray(p, jnp.float32).reshape(-1)

    out = pl.pallas_call(
        functools.partial(_gem_body, eps=1e-6, log_hw=math.log(HW),
                          hw=HW, g=G),
        out_shape=jax.ShapeDtypeStruct((R, G), jnp.float32),
        grid_spec=pltpu.PrefetchScalarGridSpec(
            num_scalar_prefetch=0,
            grid=(nb,),
            in_specs=[
                pl.BlockSpec(memory_space=pltpu.MemorySpace.SMEM),
                pl.BlockSpec((RB, G * HW), lambda i: (i, 0)),
            ],
            out_specs=pl.BlockSpec((RB, G), lambda i: (i, 0)),
        ),
        compiler_params=pltpu.CompilerParams(
            dimension_semantics=("parallel",),
        ),
    )(p_arr, x2d)

    return out.reshape(N, C, 1, 1).astype(x.dtype)


# speedup vs baseline: 1.3376x; 1.3376x over previous
import functools
import math

import jax
import jax.numpy as jnp
import numpy as np
from jax import lax
from jax.experimental import pallas as pl
from jax.experimental.pallas import tpu as pltpu


def _gem_body(p_ref, x_ref, v_ref, eq_ref, gt_ref, l_ref, o_ref, *,
              log_hw, st):
    p = p_ref[0]
    x = x_ref[...]
    xc = jnp.maximum(x, 1e-6)
    xp = jnp.exp(p * jnp.log(xc))

    Q = jax.lax.dot_general(xp, v_ref[...], (((1,), (0,)), ((), ())),
                            preferred_element_type=jnp.float32)
    rs = jnp.sum(xp, axis=1, keepdims=True)

    M = Q * eq_ref[...] + rs * gt_ref[...]
    G = jax.lax.dot_general(l_ref[...], M, (((1,), (0,)), ((), ())),
                            preferred_element_type=jnp.float32)

    lane = lax.broadcasted_iota(jnp.int32, (st, 128), 1)
    g_next = jnp.where(lane < 127, pltpu.roll(G, 127, axis=1), 0.0)
    s = G - g_next

    o_ref[...] = jnp.exp((jnp.log(s) - log_hw) / p)


def _static_tables(hw, st):
    c = np.arange(128)
    r_c = (hw * c) // 128
    a_c = hw * c - 128 * r_c
    V = (np.arange(128)[:, None] >= a_c[None, :]).astype(np.float32)
    r_loc = np.arange(st * hw) % hw
    EQ = (r_loc[:, None] == r_c[None, :]).astype(np.float32)
    GT = (r_loc[:, None] > r_c[None, :]).astype(np.float32)
    L = (np.arange(st * hw)[None, :] // hw
         == np.arange(st)[:, None]).astype(np.float32)
    return V, EQ, GT, L


def kernel(x, p):
    N, C, H, W = x.shape
    NC, HW = N * C, H * W
    F = NC * HW // 128
    x2d = x.reshape(F, 128)

    ST = 64
    while (NC // 128) % ST:
        ST //= 2
    nb = NC // 128 // ST

    p_arr = jnp.asarray(p, jnp.float32).reshape(-1)
    V, EQ, GT, L = (jnp.asarray(a) for a in _static_tables(HW, ST))
    const = lambda i: (0, 0)

    out = pl.pallas_call(
        functools.partial(_gem_body, log_hw=math.log(HW), st=ST),
        out_shape=jax.ShapeDtypeStruct((NC // 128, 128), jnp.float32),
        grid_spec=pltpu.PrefetchScalarGridSpec(
            num_scalar_prefetch=0,
            grid=(nb,),
            in_specs=[
                pl.BlockSpec(memory_space=pltpu.MemorySpace.SMEM),
                pl.BlockSpec((ST * HW, 128), lambda i: (i, 0)),
                pl.BlockSpec((128, 128), const),
                pl.BlockSpec((ST * HW, 128), const),
                pl.BlockSpec((ST * HW, 128), const),
                pl.BlockSpec((ST, ST * HW), const),
            ],
            out_specs=pl.BlockSpec((ST, 128), lambda i: (i, 0)),
        ),
        compiler_params=pltpu.CompilerParams(
            dimension_semantics=("parallel",),
            vmem_limit_bytes=100 * 1024 * 1024,
        ),
    )(p_arr, x2d, V, EQ, GT, L)

    return out.reshape(N, C, 1, 1).astype(x.dtype)


# speedup vs baseline: 33.6516x; 25.1586x over previous
import functools
import math

import jax
import jax.numpy as jnp
from jax.experimental import pallas as pl
from jax.experimental.pallas import tpu as pltpu


def _gem_body(p_ref, x_ref, o_ref, *, log_hw, nb, c):
    p = p_ref[0]
    x = x_ref[...]
    xc = jnp.maximum(x, 1e-6)
    xp = jnp.exp(p * jnp.log(xc))
    s = jnp.sum(xp, axis=0)
    r = jnp.exp((jnp.log(s) - log_hw) / p)
    o_ref[...] = r.reshape(nb * c // 128, 128)


def kernel(x, p):
    N, C, H, W = x.shape
    NC, HW = N * C, H * W
    x3d = x.reshape(N, C, HW).transpose(2, 0, 1)

    NB = 8
    while N % NB:
        NB //= 2
    nsteps = N // NB

    p_arr = jnp.asarray(p, jnp.float32).reshape(-1)

    out = pl.pallas_call(
        functools.partial(_gem_body, log_hw=math.log(HW), nb=NB, c=C),
        out_shape=jax.ShapeDtypeStruct((NC // 128, 128), jnp.float32),
        grid_spec=pltpu.PrefetchScalarGridSpec(
            num_scalar_prefetch=0,
            grid=(nsteps,),
            in_specs=[
                pl.BlockSpec(memory_space=pltpu.MemorySpace.SMEM),
                pl.BlockSpec((HW, NB, C), lambda i: (0, i, 0)),
            ],
            out_specs=pl.BlockSpec((NB * C // 128, 128), lambda i: (i, 0)),
        ),
        compiler_params=pltpu.CompilerParams(
            dimension_semantics=("parallel",),
        ),
    )(p_arr, x3d)

    return out.reshape(N, C, 1, 1).astype(x.dtype)


# speedup vs baseline: 38.6833x; 1.1495x over previous
import functools
import math

import jax
import jax.numpy as jnp
from jax.experimental import pallas as pl
from jax.experimental.pallas import tpu as pltpu


def _gem_body(p_ref, x_ref, o_ref, *, log_hw, nb, c):
    p = p_ref[0]
    x = x_ref[...]
    xc = jnp.maximum(x, 1e-6)
    xp = jnp.exp(p * jnp.log(xc))
    s = jnp.sum(xp, axis=0)
    r = jnp.exp((jnp.log(s) - log_hw) / p)
    o_ref[...] = r.reshape(nb * c // 128, 128)


def kernel(x, p):
    N, C, H, W = x.shape
    NC, HW = N * C, H * W
    x3d = x.reshape(N, C, HW).transpose(2, 0, 1)

    NB = 16
    while N % NB:
        NB //= 2
    nsteps = N // NB

    p_arr = jnp.asarray(p, jnp.float32).reshape(-1)

    out = pl.pallas_call(
        functools.partial(_gem_body, log_hw=math.log(HW), nb=NB, c=C),
        out_shape=jax.ShapeDtypeStruct((NC // 128, 128), jnp.float32),
        grid_spec=pltpu.PrefetchScalarGridSpec(
            num_scalar_prefetch=0,
            grid=(nsteps,),
            in_specs=[
                pl.BlockSpec(memory_space=pltpu.MemorySpace.SMEM),
                pl.BlockSpec((HW, NB, C), lambda i: (0, i, 0)),
            ],
            out_specs=pl.BlockSpec((NB * C // 128, 128), lambda i: (i, 0)),
        ),
        compiler_params=pltpu.CompilerParams(
            dimension_semantics=("parallel",),
        ),
    )(p_arr, x3d)

    return out.reshape(N, C, 1, 1).astype(x.dtype)


# speedup vs baseline: 39.6700x; 1.0255x over previous
import functools
import math

import jax
import jax.numpy as jnp
from jax.experimental import pallas as pl
from jax.experimental.pallas import tpu as pltpu


def _gem_body(p_ref, x_ref, o_ref, *, log_hw, nb, c):
    p = p_ref[0]
    x = x_ref[...]
    xc = jnp.maximum(x, 1e-6)
    xp = jnp.exp(p * jnp.log(xc))
    s = jnp.sum(xp, axis=0)
    r = jnp.exp((jnp.log(s) - log_hw) / p)
    o_ref[...] = r.reshape(nb * c // 128, 128)


def kernel(x, p):
    N, C, H, W = x.shape
    NC, HW = N * C, H * W
    x3d = x.reshape(N, C, HW).transpose(2, 0, 1)

    NB = 32
    while N % NB:
        NB //= 2
    nsteps = N // NB

    p_arr = jnp.asarray(p, jnp.float32).reshape(-1)

    out = pl.pallas_call(
        functools.partial(_gem_body, log_hw=math.log(HW), nb=NB, c=C),
        out_shape=jax.ShapeDtypeStruct((NC // 128, 128), jnp.float32),
        grid_spec=pltpu.PrefetchScalarGridSpec(
            num_scalar_prefetch=0,
            grid=(nsteps,),
            in_specs=[
                pl.BlockSpec(memory_space=pltpu.MemorySpace.SMEM),
                pl.BlockSpec((HW, NB, C), lambda i: (0, i, 0)),
            ],
            out_specs=pl.BlockSpec((NB * C // 128, 128), lambda i: (i, 0)),
        ),
        compiler_params=pltpu.CompilerParams(
            dimension_semantics=("parallel",),
            vmem_limit_bytes=100 * 1024 * 1024,
        ),
    )(p_arr, x3d)

    return out.reshape(N, C, 1, 1).astype(x.dtype)
